# baseline (speedup 1.0000x reference)
import functools

import jax
import jax.numpy as jnp
from jax import lax
from jax.experimental import pallas as pl
from jax.experimental.pallas import tpu as pltpu
from jax.experimental.pallas import tpu_sc as plsc

N = 10000
E = 320000
D = 128

NC = 2
NS = 16
NW = NC * NS
EPW = E // NW
K = 40
NCHUNK = EPW // K
NBUF = 4
N_PAD = 10240
RPT = N_PAD // NS

MM_BLOCK = 1000


def _mm_body(x_ref, w_ref, o_ref):
    o_ref[...] = jnp.dot(x_ref[...], w_ref[...],
                         preferred_element_type=jnp.float32)


def _combine_body(p0_ref, p1_ref, b_ref, o_ref):
    o_ref[...] = p0_ref[...] + p1_ref[...] + b_ref[...]


def _sc_body(col_hbm, row_hbm, val_hbm, sup_hbm, zero_hbm, out_hbm,
             cb0, cb1, cb2, cb3, rb0, rb1, rb2, rb3, vb0, vb1, vb2, vb3,
             gath0, gath1, gath2, gath3, scal0, scal1,
             semc0, semc1, semc2, semc3, semr0, semr1, semr2, semr3,
             semv0, semv1, semv2, semv3, semg0, semg1, semg2, semg3,
             sems0, sems1, sems2, sems3,
             acc_ref):
    c = lax.axis_index("c")
    s = lax.axis_index("s")
    wid = s * NC + c

    cb = (cb0, cb1, cb2, cb3)
    rb = (rb0, rb1, rb2, rb3)
    vb = (vb0, vb1, vb2, vb3)
    gath = (gath0, gath1, gath2, gath3)
    scal = (scal0, scal1)
    semc = (semc0, semc1, semc2, semc3)
    semr = (semr0, semr1, semr2, semr3)
    semv = (semv0, semv1, semv2, semv3)
    semg = (semg0, semg1, semg2, semg3)
    sems = (sems0, sems1, sems2, sems3)

    def col_src(g):
        return col_hbm.at[pl.ds(wid * EPW + g * K, K)]

    def row_src(g):
        return row_hbm.at[pl.ds(wid * EPW + g * K, K)]

    def val_src(g):
        return val_hbm.at[pl.ds(wid * EPW + g * K, K)]

    pltpu.sync_copy(zero_hbm, acc_ref.at[pl.ds(s * RPT, RPT)])
    plsc.subcore_barrier()

    pltpu.sync_copy(col_src(0), cb[0])
    pltpu.async_copy(col_src(1), cb[1], semc[1])
    pltpu.async_copy(col_src(2), cb[2], semc[2])
    pltpu.async_copy(col_src(3), cb[3], semc[3])
    pltpu.async_copy(val_src(0), vb[0].at[pl.ds(0, K)], semv[0])
    pltpu.async_copy(val_src(1), vb[1].at[pl.ds(0, K)], semv[1])
    pltpu.async_copy(row_src(0), rb[0], semr[0])
    pltpu.async_copy(row_src(1), rb[1], semr[1])
    pltpu.async_copy(sup_hbm.at[cb[0]], gath[0], semg[0])
    pltpu.make_async_copy(col_src(1), cb[1], semc[1]).wait()
    pltpu.async_copy(sup_hbm.at[cb[1]], gath[1], semg[1])
    pltpu.make_async_copy(col_src(2), cb[2], semc[2]).wait()
    pltpu.async_copy(sup_hbm.at[cb[2]], gath[2], semg[2])

    def emit_iter(g, b):
        b2 = (b + 2) % NBUF
        b3 = (b + 3) % NBUF
        sb = b % 2

        @pl.when(g >= 2)
        def _():
            pltpu.make_async_copy(
                scal[sb], acc_ref.at[rb[b2]], sems[b2]).wait()

        @pl.when(g + 3 < NCHUNK)
        def _():
            pltpu.make_async_copy(col_src(g + 3), cb[b3], semc[b3]).wait()
            pltpu.async_copy(sup_hbm.at[cb[b3]], gath[b3], semg[b3])

        @pl.when(g + 2 < NCHUNK)
        def _():
            pltpu.async_copy(val_src(g + 2), vb[b2].at[pl.ds(0, K)],
                             semv[b2])
            pltpu.async_copy(row_src(g + 2), rb[b2], semr[b2])

        pltpu.make_async_copy(sup_hbm.at[cb[b]], gath[b], semg[b]).wait()

        @pl.when(g + 4 < NCHUNK)
        def _():
            pltpu.async_copy(col_src(g + 4), cb[b], semc[b])

        pltpu.make_async_copy(
            val_src(g), vb[b].at[pl.ds(0, K)], semv[b]).wait()

        for eg in range((K + 15) // 16):
            vgroup = vb[b][pl.ds(eg * 16, 16)]
            for e16 in range(min(16, K - eg * 16)):
                vsc = jnp.full((16,), vgroup[e16])
                e = eg * 16 + e16
                for f in range(D // 16):
                    scal[sb][e, pl.ds(f * 16, 16)] = (
                        gath[b][e, pl.ds(f * 16, 16)] * vsc)

        pltpu.make_async_copy(row_src(g), rb[b], semr[b]).wait()
        pltpu.async_copy(scal[sb], acc_ref.at[rb[b]], sems[b], add=True)

    def quad(i, carry):
        emit_iter(4 * i, 0)
        emit_iter(4 * i + 1, 1)
        emit_iter(4 * i + 2, 2)
        emit_iter(4 * i + 3, 3)
        return carry

    lax.fori_loop(0, (NCHUNK - 2) // NBUF, quad, 0)
    emit_iter(NCHUNK - 2, 0)
    emit_iter(NCHUNK - 1, 1)
    pltpu.make_async_copy(
        scal[0], acc_ref.at[rb[0]], sems[0]).wait()
    pltpu.make_async_copy(
        scal[1], acc_ref.at[rb[1]], sems[1]).wait()
    plsc.subcore_barrier()

    pltpu.sync_copy(acc_ref.at[pl.ds(s * RPT, RPT)],
                    out_hbm.at[c, pl.ds(s * RPT, RPT)])


def kernel(x, adj_indices, adj_values, weight, bias):
    adj = adj_indices.astype(jnp.int32)
    row1 = adj[0]
    col1 = adj[1]
    val1 = adj_values
    zeros = jnp.zeros((RPT, D), jnp.float32)

    support = pl.pallas_call(
        _mm_body,
        out_shape=jax.ShapeDtypeStruct((N, D), jnp.float32),
    )(x, weight)

    buf_types = []
    for dt in (jnp.int32, jnp.int32):
        buf_types += [pltpu.VMEM((K,), dt)] * NBUF
    buf_types += [pltpu.VMEM((128,), jnp.float32)] * NBUF
    buf_types += [pltpu.VMEM((K, D), jnp.float32)] * NBUF
    buf_types += [pltpu.VMEM((K, D), jnp.float32)] * 2
    sem_types = [pltpu.SemaphoreType.DMA] * (5 * NBUF)

    sc = functools.partial(
        pl.kernel,
        mesh=plsc.VectorSubcoreMesh(core_axis_name="c", subcore_axis_name="s"),
        out_type=jax.ShapeDtypeStruct((NC, N_PAD, D), jnp.float32),
        scratch_types=buf_types + sem_types + [
            pltpu.VMEM_SHARED((N_PAD, D), jnp.float32),
        ],
    )(_sc_body)
    partials = sc(col1, row1, val1, support, zeros)

    out = pl.pallas_call(
        _combine_body,
        out_shape=jax.ShapeDtypeStruct((N, D), jnp.float32),
    )(partials[0][:N], partials[1][:N], bias.reshape(1, D))
    return out


# speedup vs baseline: 1.0226x; 1.0226x over previous
import functools

import jax
import jax.numpy as jnp
from jax import lax
from jax.experimental import pallas as pl
from jax.experimental.pallas import tpu as pltpu
from jax.experimental.pallas import tpu_sc as plsc

N = 10000
E = 320000
D = 128

NC = 2
NS = 16
NW = NC * NS
EPW = E // NW
K = 80
NCHUNK = EPW // K
NBUF = 4
QK = 4 * K
NQ = 32
EPWP = NQ * QK
N_PAD = 10240
RPT = N_PAD // NS


def _mm_body(x_ref, w_ref, o_ref):
    o_ref[...] = jnp.dot(x_ref[...], w_ref[...],
                         preferred_element_type=jnp.float32)


def _combine_body(p0_ref, p1_ref, b_ref, o_ref):
    o_ref[...] = p0_ref[...] + p1_ref[...] + b_ref[...]


def _sc_body(col_hbm, row_hbm, val_hbm, sup_hbm, zero_hbm, out_hbm,
             col4_0, col4_1, col4_2, row4_0, row4_1, row4_2,
             val4_0, val4_1, val4_2, gath0, gath1, gath2, gath3,
             sqc0, sqc1, sqc2, sqr0, sqr1, sqr2, sqv0, sqv1, sqv2,
             semg0, semg1, semg2, semg3, sems0, sems1, sems2, sems3,
             acc_ref):
    c = lax.axis_index("c")
    s = lax.axis_index("s")
    wid = s * NC + c

    col4 = (col4_0, col4_1, col4_2)
    row4 = (row4_0, row4_1, row4_2)
    val4 = (val4_0, val4_1, val4_2)
    gath = (gath0, gath1, gath2, gath3)
    sqc = (sqc0, sqc1, sqc2)
    sqr = (sqr0, sqr1, sqr2)
    sqv = (sqv0, sqv1, sqv2)
    semg = (semg0, semg1, semg2, semg3)
    sems = (sems0, sems1, sems2, sems3)

    def qcol_src(q):
        return col_hbm.at[pl.ds(wid * EPWP + q * QK, QK)]

    def qval_src(q):
        return val_hbm.at[pl.ds(wid * EPWP + q * QK, QK)]

    def qrow_src(q):
        return row_hbm.at[wid, q]

    def load_quad(q, r):
        pltpu.async_copy(qcol_src(q), col4[r], sqc[r])
        pltpu.async_copy(qrow_src(q), row4[r], sqr[r])
        pltpu.async_copy(qval_src(q), val4[r], sqv[r])

    def wait_quad(q, r):
        pltpu.make_async_copy(qcol_src(q), col4[r], sqc[r]).wait()
        pltpu.make_async_copy(qrow_src(q), row4[r], sqr[r]).wait()
        pltpu.make_async_copy(qval_src(q), val4[r], sqv[r]).wait()

    pltpu.sync_copy(zero_hbm, acc_ref.at[pl.ds(s * RPT, RPT)])
    plsc.subcore_barrier()

    pltpu.sync_copy(qcol_src(0), col4[0])
    pltpu.sync_copy(qrow_src(0), row4[0])
    pltpu.sync_copy(qval_src(0), val4[0])
    load_quad(1, 1)
    pltpu.async_copy(sup_hbm.at[col4[0].at[pl.ds(0, K)]], gath[0], semg[0])
    pltpu.async_copy(sup_hbm.at[col4[0].at[pl.ds(K, K)]], gath[1], semg[1])

    def emit_iter(g, ring, p):
        b = p
        b2 = (b + 2) % NBUF
        p2 = (p + 2) % 4
        ring2 = ring if p < 2 else (ring + 1) % 3
        ring_m2 = ring if p >= 2 else (ring - 1) % 3

        @pl.when(g >= 2)
        def _():
            pltpu.make_async_copy(
                gath[b2], acc_ref.at[row4[ring_m2].at[p2]],
                sems[b2]).wait()

        @pl.when(g + 2 < NCHUNK)
        def _():
            pltpu.async_copy(
                sup_hbm.at[col4[ring2].at[pl.ds(p2 * K, K)]],
                gath[b2], semg[b2])

        pltpu.make_async_copy(
            sup_hbm.at[col4[ring].at[pl.ds(p * K, K)]],
            gath[b], semg[b]).wait()

        def group(eg, carry2):
            vgroup = val4[ring][pl.ds(p * K + eg * 16, 16)]
            for e16 in range(16):
                vsc = jnp.full((16,), vgroup[e16])
                e = eg * 16 + e16
                for f in range(D // 16):
                    gath[b][e, pl.ds(f * 16, 16)] = (
                        gath[b][e, pl.ds(f * 16, 16)] * vsc)
            return carry2

        lax.fori_loop(0, K // 16, group, 0)

        pltpu.async_copy(gath[b], acc_ref.at[row4[ring].at[p]],
                         sems[b], add=True)

    def quad_body(i, ring):
        g0 = 4 * i
        emit_iter(g0, ring, 0)
        emit_iter(g0 + 1, ring, 1)
        load_quad(i + 2, (ring + 2) % 3)
        wait_quad(i + 1, (ring + 1) % 3)
        emit_iter(g0 + 2, ring, 2)
        emit_iter(g0 + 3, ring, 3)

    def tri_quad(j, carry):
        quad_body(3 * j, 0)
        quad_body(3 * j + 1, 1)
        quad_body(3 * j + 2, 2)
        return carry

    lax.fori_loop(0, 10, tri_quad, 0)

    emit_iter(120, 0, 0)
    emit_iter(121, 0, 1)
    wait_quad(31, 1)
    emit_iter(122, 0, 2)
    emit_iter(123, 0, 3)
    emit_iter(124, 1, 0)
    pltpu.make_async_copy(
        gath[3], acc_ref.at[row4[0].at[3]], sems[3]).wait()
    pltpu.make_async_copy(
        gath[0], acc_ref.at[row4[1].at[0]], sems[0]).wait()
    plsc.subcore_barrier()

    pltpu.sync_copy(acc_ref.at[pl.ds(s * RPT, RPT)],
                    out_hbm.at[c, pl.ds(s * RPT, RPT)])


def kernel(x, adj_indices, adj_values, weight, bias):
    adj = adj_indices.astype(jnp.int32)
    pad = ((0, 0), (0, EPWP - EPW))
    row4d = jnp.pad(adj[0].reshape(NW, EPW), pad).reshape(NW, NQ, 4, K)
    col1 = jnp.pad(adj[1].reshape(NW, EPW), pad).reshape(NW * EPWP)
    val1 = jnp.pad(adj_values.reshape(NW, EPW), pad).reshape(NW * EPWP)
    zeros = jnp.zeros((RPT, D), jnp.float32)

    support = pl.pallas_call(
        _mm_body,
        out_shape=jax.ShapeDtypeStruct((N, D), jnp.float32),
    )(x, weight)

    buf_types = [pltpu.VMEM((QK,), jnp.int32)] * 3
    buf_types += [pltpu.VMEM((4, K), jnp.int32)] * 3
    buf_types += [pltpu.VMEM((QK,), jnp.float32)] * 3
    buf_types += [pltpu.VMEM((K, D), jnp.float32)] * NBUF
    sem_types = [pltpu.SemaphoreType.DMA] * 17

    sc = functools.partial(
        pl.kernel,
        mesh=plsc.VectorSubcoreMesh(core_axis_name="c", subcore_axis_name="s"),
        out_type=jax.ShapeDtypeStruct((NC, N_PAD, D), jnp.float32),
        scratch_types=buf_types + sem_types + [
            pltpu.VMEM_SHARED((N_PAD, D), jnp.float32),
        ],
    )(_sc_body)
    partials = sc(col1, row4d, val1, support, zeros)

    out = pl.pallas_call(
        _combine_body,
        out_shape=jax.ShapeDtypeStruct((N, D), jnp.float32),
    )(partials[0][:N], partials[1][:N], bias.reshape(1, D))
    return out


# speedup vs baseline: 1.0923x; 1.0681x over previous
import functools

import jax
import jax.numpy as jnp
from jax import lax
from jax.experimental import pallas as pl
from jax.experimental.pallas import tpu as pltpu
from jax.experimental.pallas import tpu_sc as plsc

N = 10000
E = 320000
D = 128

NC = 2
NS = 16
NW = NC * NS
EPW = E // NW
K = 80
NCHUNK = EPW // K
NBUF = 4
N_PAD = 10240
RPT = N_PAD // NS

MM_BLOCK = 1000


def _mm_body(x_ref, w_ref, o_ref):
    o_ref[...] = jnp.dot(x_ref[...], w_ref[...],
                         preferred_element_type=jnp.float32)


def _combine_body(p0_ref, p1_ref, b_ref, o_ref):
    o_ref[...] = p0_ref[...] + p1_ref[...] + b_ref[...]


def _sc_body(col_hbm, row_hbm, val_hbm, sup_hbm, zero_hbm, out_hbm,
             cb0, cb1, cb2, cb3, rb0, rb1, rb2, rb3, vb0, vb1, vb2, vb3,
             gath0, gath1, gath2, gath3,
             semc0, semc1, semc2, semc3, semr0, semr1, semr2, semr3,
             semv0, semv1, semv2, semv3, semg0, semg1, semg2, semg3,
             sems0, sems1, sems2, sems3,
             acc_ref):
    c = lax.axis_index("c")
    s = lax.axis_index("s")
    wid = s * NC + c

    cb = (cb0, cb1, cb2, cb3)
    rb = (rb0, rb1, rb2, rb3)
    vb = (vb0, vb1, vb2, vb3)
    gath = (gath0, gath1, gath2, gath3)
    semc = (semc0, semc1, semc2, semc3)
    semr = (semr0, semr1, semr2, semr3)
    semv = (semv0, semv1, semv2, semv3)
    semg = (semg0, semg1, semg2, semg3)
    sems = (sems0, sems1, sems2, sems3)

    def col_src(g):
        return col_hbm.at[pl.ds(wid * EPW + g * K, K)]

    def row_src(g):
        return row_hbm.at[pl.ds(wid * EPW + g * K, K)]

    def val_src(g):
        return val_hbm.at[pl.ds(wid * EPW + g * K, K)]

    pltpu.async_copy(col_src(0), cb[0], semc[0])
    pltpu.async_copy(col_src(1), cb[1], semc[1])
    pltpu.async_copy(col_src(2), cb[2], semc[2])
    pltpu.async_copy(val_src(0), vb[0], semv[0])
    pltpu.async_copy(val_src(1), vb[1], semv[1])
    pltpu.async_copy(row_src(0), rb[0], semr[0])
    pltpu.async_copy(row_src(1), rb[1], semr[1])
    pltpu.sync_copy(zero_hbm, acc_ref.at[pl.ds(s * RPT, RPT)])
    pltpu.make_async_copy(col_src(0), cb[0], semc[0]).wait()
    pltpu.async_copy(sup_hbm.at[cb[0]], gath[0], semg[0])
    pltpu.make_async_copy(col_src(1), cb[1], semc[1]).wait()
    pltpu.async_copy(sup_hbm.at[cb[1]], gath[1], semg[1])
    plsc.subcore_barrier()

    def emit_iter(g, b):
        b2 = (b + 2) % NBUF
        b3 = (b + 3) % NBUF

        @pl.when(g >= 2)
        def _():
            pltpu.make_async_copy(
                gath[b2], acc_ref.at[rb[b2]], sems[b2]).wait()

        @pl.when(g + 2 < NCHUNK)
        def _():
            pltpu.make_async_copy(col_src(g + 2), cb[b2], semc[b2]).wait()
            pltpu.async_copy(sup_hbm.at[cb[b2]], gath[b2], semg[b2])
            pltpu.async_copy(val_src(g + 2), vb[b2], semv[b2])
            pltpu.async_copy(row_src(g + 2), rb[b2], semr[b2])

        @pl.when(g + 3 < NCHUNK)
        def _():
            pltpu.async_copy(col_src(g + 3), cb[b3], semc[b3])

        pltpu.make_async_copy(sup_hbm.at[cb[b]], gath[b], semg[b]).wait()
        pltpu.make_async_copy(val_src(g), vb[b], semv[b]).wait()

        def group(eg, carry2):
            vgroup = vb[b][pl.ds(eg * 16, 16)]
            for e16 in range(16):
                vsc = jnp.full((16,), vgroup[e16])
                e = eg * 16 + e16
                for f in range(D // 16):
                    gath[b][e, pl.ds(f * 16, 16)] = (
                        gath[b][e, pl.ds(f * 16, 16)] * vsc)
            return carry2

        lax.fori_loop(0, K // 16, group, 0)

        pltpu.make_async_copy(row_src(g), rb[b], semr[b]).wait()
        pltpu.async_copy(gath[b], acc_ref.at[rb[b]], sems[b], add=True)

    def quad(i, carry):
        emit_iter(4 * i, 0)
        emit_iter(4 * i + 1, 1)
        emit_iter(4 * i + 2, 2)
        emit_iter(4 * i + 3, 3)
        return carry

    lax.fori_loop(0, (NCHUNK - 1) // NBUF, quad, 0)
    emit_iter(NCHUNK - 1, 0)
    pltpu.make_async_copy(
        gath[3], acc_ref.at[rb[3]], sems[3]).wait()
    pltpu.make_async_copy(
        gath[0], acc_ref.at[rb[0]], sems[0]).wait()
    plsc.subcore_barrier()

    pltpu.sync_copy(acc_ref.at[pl.ds(s * RPT, RPT)],
                    out_hbm.at[c, pl.ds(s * RPT, RPT)])


def kernel(x, adj_indices, adj_values, weight, bias):
    adj = adj_indices.astype(jnp.int32)
    row1 = adj[0]
    col1 = adj[1]
    val1 = adj_values
    zeros = jnp.zeros((RPT, D), jnp.float32)

    support = pl.pallas_call(
        _mm_body,
        out_shape=jax.ShapeDtypeStruct((N, D), jnp.float32),
    )(x, weight)

    buf_types = []
    for dt in (jnp.int32, jnp.int32, jnp.float32):
        buf_types += [pltpu.VMEM((K,), dt)] * NBUF
    buf_types += [pltpu.VMEM((K, D), jnp.float32)] * NBUF
    sem_types = [pltpu.SemaphoreType.DMA] * (5 * NBUF)

    sc = functools.partial(
        pl.kernel,
        mesh=plsc.VectorSubcoreMesh(core_axis_name="c", subcore_axis_name="s"),
        out_type=jax.ShapeDtypeStruct((NC, N_PAD, D), jnp.float32),
        scratch_types=buf_types + sem_types + [
            pltpu.VMEM_SHARED((N_PAD, D), jnp.float32),
        ],
    )(_sc_body)
    partials = sc(col1, row1, val1, support, zeros)

    out = pl.pallas_call(
        _combine_body,
        grid=(1,),
        in_specs=[
            pl.BlockSpec((N, D), lambda i: (0, 0)),
            pl.BlockSpec((N, D), lambda i: (0, 0)),
            pl.BlockSpec((1, D), lambda i: (0, 0)),
        ],
        out_specs=pl.BlockSpec((N, D), lambda i: (0, 0)),
        out_shape=jax.ShapeDtypeStruct((N, D), jnp.float32),
    )(partials[0], partials[1], bias.reshape(1, D))
    return out
